# baseline (speedup 1.0000x reference)
import functools

import jax
import jax.numpy as jnp
from jax import lax
from jax.experimental import pallas as pl
from jax.experimental.pallas import tpu as pltpu
from jax.experimental.pallas import tpu_sc as plsc

_D = 64
_DP = 128
_MP = 32
_NW = 32
_CH = 128
_G = 2
_GR = _CH * _G
_NBUF = 2
_NBUF_R = 4
_L = 16


@functools.cache
def _make_relayout(nt):
    nblk = nt // _DP
    npw = (nblk + _NW - 1) // _NW
    mesh = plsc.VectorSubcoreMesh(core_axis_name="c", subcore_axis_name="s")

    @functools.partial(
        pl.kernel,
        mesh=mesh,
        out_type=jax.ShapeDtypeStruct((nt, _DP), jnp.float32),
        compiler_params=pltpu.CompilerParams(
            use_tc_tiling_on_sc=True,
            needs_layout_passes=False,
            disable_bounds_checks=True,
        ),
        scratch_types=[
            *[pltpu.VMEM((_D, _DP), jnp.float32) for _ in range(_NBUF_R)],
            *[pltpu.VMEM((_DP, _DP), jnp.float32) for _ in range(_NBUF_R)],
            *[pltpu.SemaphoreType.DMA for _ in range(2 * _NBUF_R)],
        ],
    )
    def relayout(tt_hbm, tail_hbm, out_hbm, *refs):
        inbs = refs[:_NBUF_R]
        outbs = refs[_NBUF_R : 2 * _NBUF_R]
        isems = refs[2 * _NBUF_R : 3 * _NBUF_R]
        osems = refs[3 * _NBUF_R :]
        wid = lax.axis_index("s") * 2 + lax.axis_index("c")
        iot = lax.iota(jnp.int32, _L)
        czero = iot * 0
        ntail = nt - nblk * _DP

        @pl.when(wid == _NW - 1)
        def _():
            pltpu.sync_copy(tail_hbm, out_hbm.at[pl.ds(nblk * _DP, ntail)])

        def in_copy(t, b):
            k = wid + t * _NW
            return (tt_hbm.at[:, pl.ds(k * _DP, _DP)], inbs[b], isems[b])

        def out_copy(t, b):
            k = wid + t * _NW
            return (outbs[b], out_hbm.at[pl.ds(k * _DP, _DP)], osems[b])

        def guarded(t, fn):
            @pl.when(wid + t * _NW < nblk)
            def _():
                fn()

        for b in range(_NBUF_R):
            guarded(b, lambda b=b: pltpu.async_copy(*in_copy(b, b)))

        def step(t, carry):
            for b in range(_NBUF_R):
                tt = t * _NBUF_R + b

                def body(tt=tt, b=b):
                    pltpu.make_async_copy(*in_copy(tt, b)).wait()

                    @pl.when(tt >= _NBUF_R)
                    def _():
                        pltpu.make_async_copy(*out_copy(tt - _NBUF_R, b)).wait()

                    def lbody(li, c2):
                        ridx = iot + li * _L
                        for c in range(_D):
                            v = inbs[b][c, pl.ds(li * _L, _L)]
                            plsc.store_scatter(outbs[b], [ridx, czero + c], v)
                        return c2

                    lax.fori_loop(0, _DP // _L, lbody, 0)
                    pltpu.async_copy(*out_copy(tt, b))
                    if True:
                        @pl.when(tt + _NBUF_R < npw)
                        def _():
                            @pl.when(wid + (tt + _NBUF_R) * _NW < nblk)
                            def _():
                                pltpu.async_copy(*in_copy(tt + _NBUF_R, b))

                guarded(tt, body)
            return carry

        lax.fori_loop(0, npw // _NBUF_R + (1 if npw % _NBUF_R else 0), step, 0)

        for b in range(_NBUF_R):
            pltpu.make_async_copy(*out_copy(b, b)).wait()

    return relayout


@functools.cache
def _make_gather(B):
    b_per_w = B // _NW
    nch = b_per_w // _CH
    ngrp = b_per_w // _GR
    mesh = plsc.VectorSubcoreMesh(core_axis_name="c", subcore_axis_name="s")

    @functools.partial(
        pl.kernel,
        mesh=mesh,
        out_type=jax.ShapeDtypeStruct((B, _DP), jnp.float32),
        compiler_params=pltpu.CompilerParams(use_tc_tiling_on_sc=True),
        scratch_types=[
            pltpu.VMEM((nch, _CH), jnp.int32),
            *[pltpu.VMEM((_GR, _DP), jnp.float32) for _ in range(_NBUF)],
            *[pltpu.SemaphoreType.DMA for _ in range(2 * _NBUF)],
        ],
    )
    def emb(idx_hbm, table_hbm, out_hbm, idx_v, *refs):
        rowss = refs[:_NBUF]
        gsems = refs[_NBUF : 2 * _NBUF]
        osems = refs[2 * _NBUF :]
        wid = lax.axis_index("s") * 2 + lax.axis_index("c")
        base = wid * b_per_w
        pltpu.sync_copy(idx_hbm.at[wid], idx_v)

        def fire(g, b):
            for k in range(_G):
                pltpu.async_copy(
                    table_hbm.at[idx_v.at[g * _G + k]],
                    rowss[b].at[pl.ds(k * _CH, _CH)],
                    gsems[b],
                )

        def wait_gathers(g, b):
            for k in range(_G):
                pltpu.make_async_copy(
                    table_hbm.at[idx_v.at[g * _G + k]],
                    rowss[b].at[pl.ds(k * _CH, _CH)],
                    gsems[b],
                ).wait()

        def out_copy(g, b):
            return (
                rowss[b],
                out_hbm.at[pl.ds(base + g * _GR, _GR)],
                osems[b],
            )

        for b in range(_NBUF):
            fire(b, b)

        def outer(i, carry):
            gg = i * _NBUF
            for b in range(_NBUF):
                g = gg + b
                wait_gathers(g, b)
                pltpu.async_copy(*out_copy(g, b))

                @pl.when(g + _NBUF < ngrp)
                def _():
                    pltpu.make_async_copy(*out_copy(g, b)).wait()
                    fire(g + _NBUF, b)

            return carry

        lax.fori_loop(0, ngrp // _NBUF, outer, 0)

        for b in range(_NBUF):
            g = ngrp - _NBUF + b
            pltpu.make_async_copy(*out_copy(g, b)).wait()

    return emb


def kernel(x, table):
    n, m = x.shape
    Bp = n * _MP
    x_p = jnp.pad(x.astype(jnp.int32), ((0, 0), (0, _MP - m)), mode="edge")
    idx = x_p.reshape(_NW, Bp // _NW // _CH, _CH)
    nt = table.shape[0]
    tail_p = jnp.pad(table[nt - nt % _DP :], ((0, 0), (0, _DP - _D)))
    table_p = _make_relayout(nt)(table.T, tail_p)
    out = _make_gather(Bp)(idx, table_p)
    return out.reshape(n, _MP, _DP)[:, :m, :_D]


# speedup vs baseline: 1.8943x; 1.8943x over previous
import functools

import jax
import jax.numpy as jnp
from jax import lax
from jax.experimental import pallas as pl
from jax.experimental.pallas import tpu as pltpu
from jax.experimental.pallas import tpu_sc as plsc

_D = 64
_DP = 128
_MP = 32
_NW = 32
_CH = 128
_G = 2
_GR = _CH * _G
_NBUF = 2
_NBUF_R = 4
_L = 16


@functools.cache
def _make_gather(B):
    b_per_w = B // _NW
    nch = b_per_w // _CH
    ngrp = b_per_w // _GR
    mesh = plsc.VectorSubcoreMesh(core_axis_name="c", subcore_axis_name="s")

    @functools.partial(
        pl.kernel,
        mesh=mesh,
        out_type=jax.ShapeDtypeStruct((B, _DP), jnp.float32),
        compiler_params=pltpu.CompilerParams(use_tc_tiling_on_sc=False),
        scratch_types=[
            pltpu.VMEM((nch, _CH), jnp.int32),
            *[pltpu.VMEM((_GR, _D), jnp.float32) for _ in range(_NBUF)],
            *[pltpu.SemaphoreType.DMA for _ in range(2 * _NBUF)],
        ],
    )
    def emb(idx_hbm, table_hbm, out_hbm, idx_v, *refs):
        rowss = refs[:_NBUF]
        gsems = refs[_NBUF : 2 * _NBUF]
        osems = refs[2 * _NBUF :]
        wid = lax.axis_index("s") * 2 + lax.axis_index("c")
        base = wid * b_per_w
        pltpu.sync_copy(idx_hbm.at[wid], idx_v)

        def fire(g, b):
            for k in range(_G):
                pltpu.async_copy(
                    table_hbm.at[idx_v.at[g * _G + k]],
                    rowss[b].at[pl.ds(k * _CH, _CH)],
                    gsems[b],
                )

        def wait_gathers(g, b):
            for k in range(_G):
                pltpu.make_async_copy(
                    table_hbm.at[idx_v.at[g * _G + k]],
                    rowss[b].at[pl.ds(k * _CH, _CH)],
                    gsems[b],
                ).wait()

        def out_copy(g, b):
            return (
                rowss[b],
                out_hbm.at[pl.ds(base + g * _GR, _GR), pl.ds(0, _D)],
                osems[b],
            )

        for b in range(_NBUF):
            fire(b, b)

        def outer(i, carry):
            gg = i * _NBUF
            for b in range(_NBUF):
                g = gg + b
                wait_gathers(g, b)
                pltpu.async_copy(*out_copy(g, b))

                @pl.when(g + _NBUF < ngrp)
                def _():
                    pltpu.make_async_copy(*out_copy(g, b)).wait()
                    fire(g + _NBUF, b)

            return carry

        lax.fori_loop(0, ngrp // _NBUF, outer, 0)

        for b in range(_NBUF):
            g = ngrp - _NBUF + b
            pltpu.make_async_copy(*out_copy(g, b)).wait()

    return emb


def kernel(x, table):
    n, m = x.shape
    Bp = n * _MP
    x_p = jnp.pad(x.astype(jnp.int32), ((0, 0), (0, _MP - m)), mode="edge")
    idx = x_p.reshape(_NW, Bp // _NW // _CH, _CH)
    nt = table.shape[0]
    t128 = lax.optimization_barrier(table.reshape(nt // 2, 2 * _D))
    table_v = t128.reshape(nt, _D)
    out = _make_gather(Bp)(idx, table_v)
    return out.reshape(n, _MP, _DP)[:, :m, :_D]


# speedup vs baseline: 1.9104x; 1.0085x over previous
import functools

import jax
import jax.numpy as jnp
from jax import lax
from jax.experimental import pallas as pl
from jax.experimental.pallas import tpu as pltpu
from jax.experimental.pallas import tpu_sc as plsc

_D = 64
_DP = 128
_MP = 32
_NW = 32
_CH = 128
_G = 4
_GR = _CH * _G
_NBUF = 2
_NBUF_R = 4
_L = 16


@functools.cache
def _make_gather(B):
    b_per_w = B // _NW
    nch = b_per_w // _CH
    ngrp = b_per_w // _GR
    mesh = plsc.VectorSubcoreMesh(core_axis_name="c", subcore_axis_name="s")

    @functools.partial(
        pl.kernel,
        mesh=mesh,
        out_type=jax.ShapeDtypeStruct((B, _DP), jnp.float32),
        compiler_params=pltpu.CompilerParams(use_tc_tiling_on_sc=False),
        scratch_types=[
            pltpu.VMEM((nch, _CH), jnp.int32),
            *[pltpu.VMEM((_GR, _D), jnp.float32) for _ in range(_NBUF)],
            *[pltpu.SemaphoreType.DMA for _ in range(2 * _NBUF)],
        ],
    )
    def emb(idx_hbm, table_hbm, out_hbm, idx_v, *refs):
        rowss = refs[:_NBUF]
        gsems = refs[_NBUF : 2 * _NBUF]
        osems = refs[2 * _NBUF :]
        wid = lax.axis_index("s") * 2 + lax.axis_index("c")
        base = wid * b_per_w
        pltpu.sync_copy(idx_hbm.at[wid], idx_v)

        def fire(g, b):
            for k in range(_G):
                pltpu.async_copy(
                    table_hbm.at[idx_v.at[g * _G + k]],
                    rowss[b].at[pl.ds(k * _CH, _CH)],
                    gsems[b],
                )

        def wait_gathers(g, b):
            for k in range(_G):
                pltpu.make_async_copy(
                    table_hbm.at[idx_v.at[g * _G + k]],
                    rowss[b].at[pl.ds(k * _CH, _CH)],
                    gsems[b],
                ).wait()

        def out_copy(g, b):
            return (
                rowss[b],
                out_hbm.at[pl.ds(base + g * _GR, _GR), pl.ds(0, _D)],
                osems[b],
            )

        for b in range(_NBUF):
            fire(b, b)

        def outer(i, carry):
            gg = i * _NBUF
            for b in range(_NBUF):
                g = gg + b
                wait_gathers(g, b)
                pltpu.async_copy(*out_copy(g, b))

                @pl.when(g + _NBUF < ngrp)
                def _():
                    pltpu.make_async_copy(*out_copy(g, b)).wait()
                    fire(g + _NBUF, b)

            return carry

        lax.fori_loop(0, ngrp // _NBUF, outer, 0)

        for b in range(_NBUF):
            g = ngrp - _NBUF + b
            pltpu.make_async_copy(*out_copy(g, b)).wait()

    return emb


def kernel(x, table):
    n, m = x.shape
    Bp = n * _MP
    x_p = jnp.pad(x.astype(jnp.int32), ((0, 0), (0, _MP - m)), mode="edge")
    idx = x_p.reshape(_NW, Bp // _NW // _CH, _CH)
    nt = table.shape[0]
    t128 = lax.optimization_barrier(table.reshape(nt // 2, 2 * _D))
    table_v = t128.reshape(nt, _D)
    out = _make_gather(Bp)(idx, table_v)
    return out.reshape(n, _MP, _DP)[:, :m, :_D]
